# baseline (device time: 986355 ns/iter reference)
import jax
import jax.numpy as jnp
from jax import lax
from jax.experimental import pallas as pl
from jax.experimental.pallas import tpu as pltpu

T = 1024
D = 2048
V_LOC = 16384
V_GLOB = 2 * V_LOC
N_CHUNK = 16
CHUNK = V_LOC // N_CHUNK


def kernel(x, W):
    def body(x_ref, w_hbm, out_hbm, l_hbm,
             w_buf, l_buf, m_ref, s_ref, ms_send, ms_recv,
             w_sems, st_sems, send_sems, recv_sems,
             ms_send_sem, ms_recv_sem, rd_sems, wr_sems):
        my_x = lax.axis_index("x")
        my_y = lax.axis_index("y")
        my_z = lax.axis_index("z")
        peer = (1 - my_x, my_y, my_z)

        barrier_sem = pltpu.get_barrier_semaphore()
        pl.semaphore_signal(barrier_sem, inc=1, device_id=peer,
                            device_id_type=pl.DeviceIdType.MESH)
        pl.semaphore_wait(barrier_sem, 1)

        my_base = my_x * V_LOC
        peer_base = (1 - my_x) * V_LOC

        m_ref[...] = jnp.full((T, 1), -1e30, jnp.float32)
        s_ref[...] = jnp.zeros((T, 1), jnp.float32)

        def w_copy(k):
            return pltpu.make_async_copy(
                w_hbm.at[:, pl.ds(k * CHUNK, CHUNK)],
                w_buf.at[k % 2],
                w_sems.at[k % 2],
            )

        def store_copy(k):
            return pltpu.make_async_copy(
                l_buf.at[k % 2],
                l_hbm.at[:, pl.ds(k * CHUNK, CHUNK)],
                st_sems.at[k % 2],
            )

        def chunk_rdma(k):
            return pltpu.make_async_remote_copy(
                src_ref=l_hbm.at[:, pl.ds(k * CHUNK, CHUNK)],
                dst_ref=out_hbm.at[:, pl.ds(my_base + k * CHUNK, CHUNK)],
                send_sem=send_sems.at[k],
                recv_sem=recv_sems.at[k],
                device_id=peer,
                device_id_type=pl.DeviceIdType.MESH,
            )

        w_copy(0).start()

        def mm_body(k, carry):
            slot = k % 2

            @pl.when(k + 1 < N_CHUNK)
            def _():
                w_copy(k + 1).start()

            w_copy(k).wait()

            logits = jnp.dot(x_ref[...], w_buf[slot],
                             preferred_element_type=jnp.float32)
            m_old = m_ref[...]
            m_new = jnp.maximum(m_old,
                                jnp.max(logits, axis=1, keepdims=True))
            e_sum = jnp.sum(jnp.exp(logits - m_new), axis=1, keepdims=True)
            s_ref[...] = s_ref[...] * jnp.exp(m_old - m_new) + e_sum
            m_ref[...] = m_new
            l_buf[slot] = logits

            store_copy(k).start()
            store_copy(k).wait()
            chunk_rdma(k).start()
            return carry

        lax.fori_loop(0, N_CHUNK, mm_body, 0)

        ms_send[:, 0:1] = m_ref[...]
        ms_send[:, 1:2] = s_ref[...]
        ms_rdma = pltpu.make_async_remote_copy(
            src_ref=ms_send,
            dst_ref=ms_recv,
            send_sem=ms_send_sem,
            recv_sem=ms_recv_sem,
            device_id=peer,
            device_id_type=pl.DeviceIdType.MESH,
        )
        ms_rdma.start()
        ms_rdma.wait()

        m_loc = m_ref[...]
        s_loc = s_ref[...]
        m_rem = ms_recv[:, 0:1]
        s_rem = ms_recv[:, 1:2]
        m_g = jnp.maximum(m_loc, m_rem)
        s_g = s_loc * jnp.exp(m_loc - m_g) + s_rem * jnp.exp(m_rem - m_g)
        inv_s = 1.0 / s_g

        n_buf = l_buf

        def normalize(src_col, dst_col, gate, n_iters):
            def nm_body(j, carry):
                slot = j % 2
                gate(j)

                @pl.when(j >= 2)
                def _():
                    pltpu.make_async_copy(
                        n_buf.at[slot],
                        out_hbm.at[:, pl.ds(dst_col(j - 2), CHUNK)],
                        wr_sems.at[slot],
                    ).wait()

                rd = pltpu.make_async_copy(
                    src_col(j),
                    n_buf.at[slot],
                    rd_sems.at[slot],
                )
                rd.start()
                rd.wait()
                n_buf[slot] = jnp.exp(n_buf[slot] - m_g) * inv_s
                pltpu.make_async_copy(
                    n_buf.at[slot],
                    out_hbm.at[:, pl.ds(dst_col(j), CHUNK)],
                    wr_sems.at[slot],
                ).start()
                return carry

            lax.fori_loop(0, n_iters, nm_body, 0)
            for j in (n_iters - 2, n_iters - 1):
                pltpu.make_async_copy(
                    n_buf.at[j % 2],
                    out_hbm.at[:, pl.ds(dst_col(j), CHUNK)],
                    wr_sems.at[j % 2],
                ).wait()

        normalize(
            src_col=lambda j: l_hbm.at[:, pl.ds(j * CHUNK, CHUNK)],
            dst_col=lambda j: my_base + j * CHUNK,
            gate=lambda j: None,
            n_iters=N_CHUNK,
        )

        def peer_gate(j):
            chunk_rdma(j).wait_recv()

        normalize(
            src_col=lambda j: out_hbm.at[:, pl.ds(peer_base + j * CHUNK,
                                                  CHUNK)],
            dst_col=lambda j: peer_base + j * CHUNK,
            gate=peer_gate,
            n_iters=N_CHUNK,
        )

        def drain_body(k, carry):
            chunk_rdma(k).wait_send()
            return carry

        lax.fori_loop(0, N_CHUNK, drain_body, 0)

    out, _ = pl.pallas_call(
        body,
        out_shape=(
            jax.ShapeDtypeStruct((T, V_GLOB), jnp.float32),
            jax.ShapeDtypeStruct((T, V_LOC), jnp.float32),
        ),
        in_specs=[
            pl.BlockSpec(memory_space=pltpu.VMEM),
            pl.BlockSpec(memory_space=pltpu.MemorySpace.HBM),
        ],
        out_specs=(
            pl.BlockSpec(memory_space=pltpu.MemorySpace.HBM),
            pl.BlockSpec(memory_space=pltpu.MemorySpace.HBM),
        ),
        scratch_shapes=[
            pltpu.VMEM((2, D, CHUNK), jnp.float32),
            pltpu.VMEM((2, T, CHUNK), jnp.float32),
            pltpu.VMEM((T, 1), jnp.float32),
            pltpu.VMEM((T, 1), jnp.float32),
            pltpu.VMEM((T, 8), jnp.float32),
            pltpu.VMEM((T, 8), jnp.float32),
            pltpu.SemaphoreType.DMA((2,)),
            pltpu.SemaphoreType.DMA((2,)),
            pltpu.SemaphoreType.DMA((N_CHUNK,)),
            pltpu.SemaphoreType.DMA((N_CHUNK,)),
            pltpu.SemaphoreType.DMA,
            pltpu.SemaphoreType.DMA,
            pltpu.SemaphoreType.DMA((2,)),
            pltpu.SemaphoreType.DMA((2,)),
        ],
        compiler_params=pltpu.CompilerParams(collective_id=0),
    )(x, W)
    return out


# device time: 837158 ns/iter; 1.1782x vs baseline; 1.1782x over previous
import jax
import jax.numpy as jnp
from jax import lax
from jax.experimental import pallas as pl
from jax.experimental.pallas import tpu as pltpu

T = 1024
D = 2048
V_LOC = 16384
V_GLOB = 2 * V_LOC
N_CHUNK = 16
CHUNK = V_LOC // N_CHUNK
HOLD = 3


def kernel(x, W):
    def body(x_ref, w_hbm, out_hbm, l_hbm,
             w_buf, l_buf, m_ref, s_ref, ms_send, ms_recv,
             w_sems, st_sems, send_sems, recv_sems,
             ms_send_sem, ms_recv_sem, rd_sems, wr_sems):
        my_x = lax.axis_index("x")
        my_y = lax.axis_index("y")
        my_z = lax.axis_index("z")
        peer = (1 - my_x, my_y, my_z)

        barrier_sem = pltpu.get_barrier_semaphore()
        pl.semaphore_signal(barrier_sem, inc=1, device_id=peer,
                            device_id_type=pl.DeviceIdType.MESH)
        pl.semaphore_wait(barrier_sem, 1)

        my_base = my_x * V_LOC
        peer_base = (1 - my_x) * V_LOC

        m_ref[...] = jnp.full((T, 1), -1e30, jnp.float32)
        s_ref[...] = jnp.zeros((T, 1), jnp.float32)

        def w_copy(k):
            return pltpu.make_async_copy(
                w_hbm.at[:, pl.ds(k * CHUNK, CHUNK)],
                w_buf.at[k % 2],
                w_sems.at[k % 2],
            )

        def store_copy(k):
            return pltpu.make_async_copy(
                l_buf.at[k % 2],
                l_hbm.at[:, pl.ds(k * CHUNK, CHUNK)],
                st_sems.at[k % 2],
            )

        def chunk_rdma(k):
            return pltpu.make_async_remote_copy(
                src_ref=l_hbm.at[:, pl.ds(k * CHUNK, CHUNK)],
                dst_ref=out_hbm.at[:, pl.ds(my_base + k * CHUNK, CHUNK)],
                send_sem=send_sems.at[k],
                recv_sem=recv_sems.at[k],
                device_id=peer,
                device_id_type=pl.DeviceIdType.MESH,
            )

        w_copy(0).start()

        def mm_body(k, carry):
            slot = k % 2

            @pl.when(k + 1 < N_CHUNK)
            def _():
                w_copy(k + 1).start()

            w_copy(k).wait()

            logits = jnp.dot(x_ref[...], w_buf[slot],
                             preferred_element_type=jnp.float32)
            m_old = m_ref[...]
            m_new = jnp.maximum(m_old,
                                jnp.max(logits, axis=1, keepdims=True))
            e_sum = jnp.sum(jnp.exp(logits - m_new), axis=1, keepdims=True)
            s_ref[...] = s_ref[...] * jnp.exp(m_old - m_new) + e_sum
            m_ref[...] = m_new
            l_buf[slot] = logits

            store_copy(k).start()
            store_copy(k).wait()

            @pl.when(k < HOLD)
            def _():
                chunk_rdma(k).start()

            return carry

        lax.fori_loop(0, N_CHUNK, mm_body, 0)

        ms_send[:, 0:1] = m_ref[...]
        ms_send[:, 1:2] = s_ref[...]
        ms_rdma = pltpu.make_async_remote_copy(
            src_ref=ms_send,
            dst_ref=ms_recv,
            send_sem=ms_send_sem,
            recv_sem=ms_recv_sem,
            device_id=peer,
            device_id_type=pl.DeviceIdType.MESH,
        )
        ms_rdma.start()

        def send_body(k, carry):
            chunk_rdma(k).start()
            return carry

        lax.fori_loop(HOLD, N_CHUNK, send_body, 0)
        ms_rdma.wait()

        m_loc = m_ref[...]
        s_loc = s_ref[...]
        m_rem = ms_recv[:, 0:1]
        s_rem = ms_recv[:, 1:2]
        m_g = jnp.maximum(m_loc, m_rem)
        s_g = s_loc * jnp.exp(m_loc - m_g) + s_rem * jnp.exp(m_rem - m_g)
        inv_s = 1.0 / s_g

        n_buf = l_buf

        def normalize(src_col, dst_col, gate, n_iters):
            def nm_body(j, carry):
                slot = j % 2
                gate(j)

                @pl.when(j >= 2)
                def _():
                    pltpu.make_async_copy(
                        n_buf.at[slot],
                        out_hbm.at[:, pl.ds(dst_col(j - 2), CHUNK)],
                        wr_sems.at[slot],
                    ).wait()

                rd = pltpu.make_async_copy(
                    src_col(j),
                    n_buf.at[slot],
                    rd_sems.at[slot],
                )
                rd.start()
                rd.wait()
                n_buf[slot] = jnp.exp(n_buf[slot] - m_g) * inv_s
                pltpu.make_async_copy(
                    n_buf.at[slot],
                    out_hbm.at[:, pl.ds(dst_col(j), CHUNK)],
                    wr_sems.at[slot],
                ).start()
                return carry

            lax.fori_loop(0, n_iters, nm_body, 0)
            for j in (n_iters - 2, n_iters - 1):
                pltpu.make_async_copy(
                    n_buf.at[j % 2],
                    out_hbm.at[:, pl.ds(dst_col(j), CHUNK)],
                    wr_sems.at[j % 2],
                ).wait()

        normalize(
            src_col=lambda j: l_hbm.at[:, pl.ds(j * CHUNK, CHUNK)],
            dst_col=lambda j: my_base + j * CHUNK,
            gate=lambda j: None,
            n_iters=N_CHUNK,
        )

        def peer_gate(j):
            chunk_rdma(j).wait_recv()

        normalize(
            src_col=lambda j: out_hbm.at[:, pl.ds(peer_base + j * CHUNK,
                                                  CHUNK)],
            dst_col=lambda j: peer_base + j * CHUNK,
            gate=peer_gate,
            n_iters=N_CHUNK,
        )

        def drain_body(k, carry):
            chunk_rdma(k).wait_send()
            return carry

        lax.fori_loop(0, N_CHUNK, drain_body, 0)

    out, _ = pl.pallas_call(
        body,
        out_shape=(
            jax.ShapeDtypeStruct((T, V_GLOB), jnp.float32),
            jax.ShapeDtypeStruct((T, V_LOC), jnp.float32),
        ),
        in_specs=[
            pl.BlockSpec(memory_space=pltpu.VMEM),
            pl.BlockSpec(memory_space=pltpu.MemorySpace.HBM),
        ],
        out_specs=(
            pl.BlockSpec(memory_space=pltpu.MemorySpace.HBM),
            pl.BlockSpec(memory_space=pltpu.MemorySpace.HBM),
        ),
        scratch_shapes=[
            pltpu.VMEM((2, D, CHUNK), jnp.float32),
            pltpu.VMEM((2, T, CHUNK), jnp.float32),
            pltpu.VMEM((T, 1), jnp.float32),
            pltpu.VMEM((T, 1), jnp.float32),
            pltpu.VMEM((T, 8), jnp.float32),
            pltpu.VMEM((T, 8), jnp.float32),
            pltpu.SemaphoreType.DMA((2,)),
            pltpu.SemaphoreType.DMA((2,)),
            pltpu.SemaphoreType.DMA((N_CHUNK,)),
            pltpu.SemaphoreType.DMA((N_CHUNK,)),
            pltpu.SemaphoreType.DMA,
            pltpu.SemaphoreType.DMA,
            pltpu.SemaphoreType.DMA((2,)),
            pltpu.SemaphoreType.DMA((2,)),
        ],
        compiler_params=pltpu.CompilerParams(collective_id=0),
    )(x, W)
    return out


# device time: 837030 ns/iter; 1.1784x vs baseline; 1.0002x over previous
import jax
import jax.numpy as jnp
from jax import lax
from jax.experimental import pallas as pl
from jax.experimental.pallas import tpu as pltpu

T = 1024
D = 2048
V_LOC = 16384
V_GLOB = 2 * V_LOC
N_CHUNK = 16
CHUNK = V_LOC // N_CHUNK
HOLD = 3


def kernel(x, W):
    def body(x_ref, w_hbm, out_hbm, l_hbm, r_hbm,
             w_buf, l_buf, m_ref, s_ref, ms_send, ms_recv,
             w_sems, st_sems, send_sems, recv_sems,
             ms_send_sem, ms_recv_sem, rd_sems, wr_sems):
        my_x = lax.axis_index("x")
        my_y = lax.axis_index("y")
        my_z = lax.axis_index("z")
        peer = (1 - my_x, my_y, my_z)

        barrier_sem = pltpu.get_barrier_semaphore()
        pl.semaphore_signal(barrier_sem, inc=1, device_id=peer,
                            device_id_type=pl.DeviceIdType.MESH)
        pl.semaphore_wait(barrier_sem, 1)

        my_base = my_x * V_LOC
        peer_base = (1 - my_x) * V_LOC

        m_ref[...] = jnp.full((T, 1), -1e30, jnp.float32)
        s_ref[...] = jnp.zeros((T, 1), jnp.float32)

        def w_copy(k):
            return pltpu.make_async_copy(
                w_hbm.at[:, pl.ds(k * CHUNK, CHUNK)],
                w_buf.at[k % 2],
                w_sems.at[k % 2],
            )

        def store_copy(k):
            return pltpu.make_async_copy(
                l_buf.at[k % 2],
                l_hbm.at[:, pl.ds(k * CHUNK, CHUNK)],
                st_sems.at[k % 2],
            )

        def chunk_rdma(k):
            return pltpu.make_async_remote_copy(
                src_ref=l_hbm.at[:, pl.ds(k * CHUNK, CHUNK)],
                dst_ref=r_hbm.at[:, pl.ds(k * CHUNK, CHUNK)],
                send_sem=send_sems.at[k],
                recv_sem=recv_sems.at[k],
                device_id=peer,
                device_id_type=pl.DeviceIdType.MESH,
            )

        w_copy(0).start()

        def mm_body(k, carry):
            slot = k % 2

            @pl.when(k + 1 < N_CHUNK)
            def _():
                w_copy(k + 1).start()

            w_copy(k).wait()

            logits = jnp.dot(x_ref[...], w_buf[slot],
                             preferred_element_type=jnp.float32)
            m_old = m_ref[...]
            m_new = jnp.maximum(m_old,
                                jnp.max(logits, axis=1, keepdims=True))
            e_sum = jnp.sum(jnp.exp(logits - m_new), axis=1, keepdims=True)
            s_ref[...] = s_ref[...] * jnp.exp(m_old - m_new) + e_sum
            m_ref[...] = m_new
            l_buf[slot] = logits

            store_copy(k).start()
            store_copy(k).wait()

            @pl.when(k < HOLD)
            def _():
                chunk_rdma(k).start()

            return carry

        lax.fori_loop(0, N_CHUNK, mm_body, 0)

        ms_send[:, 0:1] = m_ref[...]
        ms_send[:, 1:2] = s_ref[...]
        ms_rdma = pltpu.make_async_remote_copy(
            src_ref=ms_send,
            dst_ref=ms_recv,
            send_sem=ms_send_sem,
            recv_sem=ms_recv_sem,
            device_id=peer,
            device_id_type=pl.DeviceIdType.MESH,
        )
        ms_rdma.start()

        def send_body(k, carry):
            chunk_rdma(k).start()
            return carry

        lax.fori_loop(HOLD, N_CHUNK, send_body, 0)
        ms_rdma.wait()

        m_loc = m_ref[...]
        s_loc = s_ref[...]
        m_rem = ms_recv[:, 0:1]
        s_rem = ms_recv[:, 1:2]
        m_g = jnp.maximum(m_loc, m_rem)
        s_g = s_loc * jnp.exp(m_loc - m_g) + s_rem * jnp.exp(m_rem - m_g)
        inv_s = 1.0 / s_g

        n_buf = l_buf

        def normalize(src_col, dst_col, gate, n_iters):
            def nm_body(j, carry):
                slot = j % 2
                gate(j)

                @pl.when(j >= 2)
                def _():
                    pltpu.make_async_copy(
                        n_buf.at[slot],
                        out_hbm.at[:, pl.ds(dst_col(j - 2), CHUNK)],
                        wr_sems.at[slot],
                    ).wait()

                rd = pltpu.make_async_copy(
                    src_col(j),
                    n_buf.at[slot],
                    rd_sems.at[slot],
                )
                rd.start()
                rd.wait()
                n_buf[slot] = jnp.exp(n_buf[slot] - m_g) * inv_s
                pltpu.make_async_copy(
                    n_buf.at[slot],
                    out_hbm.at[:, pl.ds(dst_col(j), CHUNK)],
                    wr_sems.at[slot],
                ).start()
                return carry

            lax.fori_loop(0, n_iters, nm_body, 0)
            for j in (n_iters - 2, n_iters - 1):
                pltpu.make_async_copy(
                    n_buf.at[j % 2],
                    out_hbm.at[:, pl.ds(dst_col(j), CHUNK)],
                    wr_sems.at[j % 2],
                ).wait()

        normalize(
            src_col=lambda j: l_hbm.at[:, pl.ds(j * CHUNK, CHUNK)],
            dst_col=lambda j: my_base + j * CHUNK,
            gate=lambda j: None,
            n_iters=N_CHUNK,
        )

        def peer_gate(j):
            chunk_rdma(j).wait_recv()

        normalize(
            src_col=lambda j: r_hbm.at[:, pl.ds(j * CHUNK, CHUNK)],
            dst_col=lambda j: peer_base + j * CHUNK,
            gate=peer_gate,
            n_iters=N_CHUNK,
        )

        def drain_body(k, carry):
            chunk_rdma(k).wait_send()
            return carry

        lax.fori_loop(0, N_CHUNK, drain_body, 0)

    out, _, _ = pl.pallas_call(
        body,
        out_shape=(
            jax.ShapeDtypeStruct((T, V_GLOB), jnp.float32),
            jax.ShapeDtypeStruct((T, V_LOC), jnp.float32),
            jax.ShapeDtypeStruct((T, V_LOC), jnp.float32),
        ),
        in_specs=[
            pl.BlockSpec(memory_space=pltpu.VMEM),
            pl.BlockSpec(memory_space=pltpu.MemorySpace.HBM),
        ],
        out_specs=(
            pl.BlockSpec(memory_space=pltpu.MemorySpace.HBM),
            pl.BlockSpec(memory_space=pltpu.MemorySpace.HBM),
            pl.BlockSpec(memory_space=pltpu.MemorySpace.HBM),
        ),
        scratch_shapes=[
            pltpu.VMEM((2, D, CHUNK), jnp.float32),
            pltpu.VMEM((2, T, CHUNK), jnp.float32),
            pltpu.VMEM((T, 1), jnp.float32),
            pltpu.VMEM((T, 1), jnp.float32),
            pltpu.VMEM((T, 8), jnp.float32),
            pltpu.VMEM((T, 8), jnp.float32),
            pltpu.SemaphoreType.DMA((2,)),
            pltpu.SemaphoreType.DMA((2,)),
            pltpu.SemaphoreType.DMA((N_CHUNK,)),
            pltpu.SemaphoreType.DMA((N_CHUNK,)),
            pltpu.SemaphoreType.DMA,
            pltpu.SemaphoreType.DMA,
            pltpu.SemaphoreType.DMA((2,)),
            pltpu.SemaphoreType.DMA((2,)),
        ],
        compiler_params=pltpu.CompilerParams(collective_id=0),
    )(x, W)
    return out
